# baseline (speedup 1.0000x reference)
import functools

import jax
import jax.numpy as jnp
from jax import lax
from jax.experimental import pallas as pl
from jax.experimental.layout import Layout, with_layout_constraint
from jax.experimental.pallas import tpu as pltpu
from jax.experimental.pallas import tpu_sc as plsc

VS = 1000000
ED = 32
NCTX = 5
B = 16384

NC = 2
NS = 16
NW = NC * NS
BPW = B // NW
CPW = BPW * NCTX
LANES = 16
NSLOT = BPW + CPW
TAIL = (VS // 1024) * 1024


def _sc_body(tgt_hbm, ctx_hbm, ttT_hbm, ctT_hbm, out_hbm,
             tidx, cidx, tbase, cbase_off, ttail, ctail, ptb, pcb,
             vals, outv, sem):
    wid = lax.axis_index("s") * NC + lax.axis_index("c")
    tstart = wid * BPW
    cstart = wid * CPW

    pltpu.sync_copy(tgt_hbm.at[pl.ds(tstart, BPW)], tidx)
    pltpu.sync_copy(ctx_hbm.at[pl.ds(cstart, CPW)], cidx)

    def _base(v):
        return ((v >> 10) << 13) + (v & 1023)

    def _tailfix(v):
        return jnp.where(v >= TAIL, jnp.int32(448), jnp.int32(0))

    for j in range(BPW // LANES):
        sl = pl.ds(j * LANES, LANES)
        v = tidx[sl]
        tbase[sl] = _base(v)
        ttail[sl] = _tailfix(v)
    for j in range(CPW // LANES):
        sl = pl.ds(j * LANES, LANES)
        v = cidx[sl]
        cbase_off[sl] = _base(v)
        ctail[sl] = _tailfix(v)

    def plane_body(e, carry):
        g = e // 8
        r = e % 8
        cconst = g * 8000000 + r * 1024
        for j in range(BPW // LANES):
            sl = pl.ds(j * LANES, LANES)
            ptb[sl] = tbase[sl] + (cconst - ttail[sl] * r)
        for j in range(CPW // LANES):
            sl = pl.ds(j * LANES, LANES)
            pcb[sl] = cbase_off[sl] + (cconst - ctail[sl] * r)
        tcp = pltpu.make_async_copy(
            ttT_hbm.at[0].at[ptb], vals.at[e, pl.ds(0, BPW)], sem)
        ccp = pltpu.make_async_copy(
            ctT_hbm.at[0].at[pcb], vals.at[e, pl.ds(BPW, CPW)], sem)
        tcp.start()
        ccp.start()
        tcp.wait()
        ccp.wait()
        return carry

    lax.fori_loop(0, ED, plane_body, 0)

    iota = lax.broadcasted_iota(jnp.int32, (LANES,), 0)

    def tile_body(t, carry):
        rows = t * LANES + iota
        accs = [jnp.zeros((LANES,), jnp.float32) for _ in range(NCTX)]
        pair0 = rows * NCTX
        for e in range(ED):
            e_vec = jnp.full((LANES,), e, jnp.int32)
            we = plsc.load_gather(vals, [e_vec, rows])
            for c in range(NCTX):
                ce = plsc.load_gather(vals, [e_vec, BPW + pair0 + c])
                accs[c] = accs[c] + we * ce
        for c in range(NCTX):
            plsc.store_scatter(outv, [pair0 + c], accs[c])
        return carry

    lax.fori_loop(0, BPW // LANES, tile_body, 0)

    pltpu.sync_copy(outv, out_hbm.at[pl.ds(cstart, CPW)])


@jax.jit
def _sc_call(tgt_flat, ctx_flat, tt_T, ct_T):
    mesh = plsc.VectorSubcoreMesh(core_axis_name="c", subcore_axis_name="s")
    fn = functools.partial(
        pl.kernel, mesh=mesh,
        out_type=jax.ShapeDtypeStruct((B * NCTX,), jnp.float32),
        scratch_types=[
            pltpu.VMEM((BPW,), jnp.int32),
            pltpu.VMEM((CPW,), jnp.int32),
            pltpu.VMEM((BPW,), jnp.int32),
            pltpu.VMEM((CPW,), jnp.int32),
            pltpu.VMEM((BPW,), jnp.int32),
            pltpu.VMEM((CPW,), jnp.int32),
            pltpu.VMEM((BPW,), jnp.int32),
            pltpu.VMEM((CPW,), jnp.int32),
            pltpu.VMEM((ED, NSLOT), jnp.float32),
            pltpu.VMEM((CPW,), jnp.float32),
            pltpu.SemaphoreType.DMA,
        ],
        compiler_params=pltpu.CompilerParams(
            needs_layout_passes=False, use_tc_tiling_on_sc=False),
    )(_sc_body)
    return fn(tgt_flat, ctx_flat, tt_T, ct_T)


def kernel(target, context, target_table, context_table):
    tgt_flat = target.reshape(B)
    ctx_flat = context.reshape(B * NCTX)
    sc_fmt = Layout(major_to_minor=(0, 1), tiling=((8,),))
    tt_T = with_layout_constraint(target_table.T, sc_fmt)
    ct_T = with_layout_constraint(context_table.T, sc_fmt)
    out_flat = _sc_call(tgt_flat, ctx_flat, tt_T, ct_T)
    return out_flat.reshape(B, NCTX)


# speedup vs baseline: 1.0887x; 1.0887x over previous
import functools

import jax
import jax.numpy as jnp
from jax import lax
from jax.experimental import pallas as pl
from jax.experimental.layout import Layout, with_layout_constraint
from jax.experimental.pallas import tpu as pltpu
from jax.experimental.pallas import tpu_sc as plsc

VS = 1000000
ED = 32
NCTX = 5
B = 16384

NC = 2
NS = 16
NW = NC * NS
BPW = B // NW
CPW = BPW * NCTX
LANES = 16
NSLOT = BPW + CPW


def _sc_body(tgt_hbm, ctx_hbm, ttT_hbm, ctT_hbm, out_hbm,
             tidx, cidx, tbase, cbase_off, ptb, pcb, ptb2, pcb2,
             vals, outv, sem, sem2):
    wid = lax.axis_index("s") * NC + lax.axis_index("c")
    tstart = wid * BPW
    cstart = wid * CPW

    pltpu.sync_copy(tgt_hbm.at[pl.ds(tstart, BPW)], tidx)
    pltpu.sync_copy(ctx_hbm.at[pl.ds(cstart, CPW)], cidx)

    def _base(v):
        return ((v >> 7) << 10) + (v & 127)

    for j in range(BPW // LANES):
        sl = pl.ds(j * LANES, LANES)
        tbase[sl] = _base(tidx[sl])
    for j in range(CPW // LANES):
        sl = pl.ds(j * LANES, LANES)
        cbase_off[sl] = _base(cidx[sl])

    def _build(e, tdst, cdst):
        g = e // 8
        r = e % 8
        cconst = g * 8000512 + r * 128
        for j in range(BPW // LANES):
            sl = pl.ds(j * LANES, LANES)
            tdst[sl] = tbase[sl] + cconst
        for j in range(CPW // LANES):
            sl = pl.ds(j * LANES, LANES)
            cdst[sl] = cbase_off[sl] + cconst

    def _fire(e, tsrc, csrc, dsem):
        pltpu.make_async_copy(
            ttT_hbm.at[0].at[tsrc], vals.at[e, pl.ds(0, BPW)], dsem).start()
        pltpu.make_async_copy(
            ctT_hbm.at[0].at[csrc], vals.at[e, pl.ds(BPW, CPW)], dsem).start()

    def _drain(e, tsrc, csrc, dsem):
        pltpu.make_async_copy(
            ttT_hbm.at[0].at[tsrc], vals.at[e, pl.ds(0, BPW)], dsem).wait()
        pltpu.make_async_copy(
            ctT_hbm.at[0].at[csrc], vals.at[e, pl.ds(BPW, CPW)], dsem).wait()

    _build(0, ptb, pcb)
    _fire(0, ptb, pcb, sem)
    _build(1, ptb2, pcb2)
    _fire(1, ptb2, pcb2, sem2)

    def plane_pair(h, carry):
        e = 2 * h
        _drain(e, ptb, pcb, sem)
        _build(e + 2, ptb, pcb)

        @pl.when(h < ED // 2 - 1)
        def _():
            _fire(e + 2, ptb, pcb, sem)

        _drain(e + 1, ptb2, pcb2, sem2)
        _build(e + 3, ptb2, pcb2)

        @pl.when(h < ED // 2 - 1)
        def _():
            _fire(e + 3, ptb2, pcb2, sem2)

        return carry

    lax.fori_loop(0, ED // 2, plane_pair, 0)

    iota = lax.broadcasted_iota(jnp.int32, (LANES,), 0)

    def tile_body(t, carry):
        rows = t * LANES + iota
        accs = [jnp.zeros((LANES,), jnp.float32) for _ in range(NCTX)]
        pair0 = rows * NCTX
        for e in range(ED):
            e_vec = jnp.full((LANES,), e, jnp.int32)
            we = plsc.load_gather(vals, [e_vec, rows])
            for c in range(NCTX):
                ce = plsc.load_gather(vals, [e_vec, BPW + pair0 + c])
                accs[c] = accs[c] + we * ce
        for c in range(NCTX):
            plsc.store_scatter(outv, [pair0 + c], accs[c])
        return carry

    lax.fori_loop(0, BPW // LANES, tile_body, 0)

    pltpu.sync_copy(outv, out_hbm.at[pl.ds(cstart, CPW)])


@jax.jit
def _sc_call(tgt_flat, ctx_flat, tt_T, ct_T):
    mesh = plsc.VectorSubcoreMesh(core_axis_name="c", subcore_axis_name="s")
    fn = functools.partial(
        pl.kernel, mesh=mesh,
        out_type=jax.ShapeDtypeStruct((B * NCTX,), jnp.float32),
        scratch_types=[
            pltpu.VMEM((BPW,), jnp.int32),
            pltpu.VMEM((CPW,), jnp.int32),
            pltpu.VMEM((BPW,), jnp.int32),
            pltpu.VMEM((CPW,), jnp.int32),
            pltpu.VMEM((BPW,), jnp.int32),
            pltpu.VMEM((CPW,), jnp.int32),
            pltpu.VMEM((BPW,), jnp.int32),
            pltpu.VMEM((CPW,), jnp.int32),
            pltpu.VMEM((ED, NSLOT), jnp.float32),
            pltpu.VMEM((CPW,), jnp.float32),
            pltpu.SemaphoreType.DMA,
            pltpu.SemaphoreType.DMA,
        ],
        compiler_params=pltpu.CompilerParams(
            needs_layout_passes=False, use_tc_tiling_on_sc=False),
    )(_sc_body)
    return fn(tgt_flat, ctx_flat, tt_T, ct_T)


def kernel(target, context, target_table, context_table):
    tgt_flat = target.reshape(B)
    ctx_flat = context.reshape(B * NCTX)
    sc_fmt = Layout(major_to_minor=(0, 1), tiling=((8,),))
    tt_T = with_layout_constraint(target_table.T, sc_fmt)
    ct_T = with_layout_constraint(context_table.T, sc_fmt)
    out_flat = _sc_call(tgt_flat, ctx_flat, tt_T, ct_T)
    return out_flat.reshape(B, NCTX)
